# baseline (speedup 1.0000x reference)
import functools

import jax
import jax.numpy as jnp
from jax import lax
from jax.experimental import pallas as pl
from jax.experimental.pallas import tpu as pltpu
from jax.experimental.pallas import tpu_sc as plsc

N = 10000
E = 10000
NNZ = 320000
DF = 128

NCORES = 2
NSUB = 16
NW = NCORES * NSUB
CHUNK = 128
NHALF = 40
FAST_CID = 0
CH_F = 160
CH_S = 0
NST_F = CH_F // NHALF
NST_S = CH_S // NHALF
TOTCH = NSUB * (CH_F + CH_S)
NNZ_PAD = TOTCH * CHUNK
SEG = 10240
JUNK = N
SEG_T = SEG // NSUB
LAST = NHALF - 1


def _tile_layout(cid, sid):
    fast = cid == FAST_CID
    base = jnp.where(fast, sid * CH_F, NSUB * CH_F + sid * CH_S)
    nst = jnp.where(fast, NST_F, NST_S)
    return base, nst


def _zero_rows(buf):
    def _zrow(i, c):
        def _zcol(g, c2):
            buf[i, pl.ds(g * 16, 16)] = jnp.zeros((16,), jnp.float32)
            return c2
        return lax.fori_loop(0, DF // 16, _zcol, c)
    lax.fori_loop(0, CHUNK, _zrow, 0)


def _fill_vec(buf, n, value):
    def _zb(g, c):
        buf[pl.ds(g * 16, 16)] = jnp.full((16,), value, jnp.float32)
        return c
    lax.fori_loop(0, n // 16, _zb, 0)


def _zero_acc(acc, buf0, sid):
    def _zacc(j, c):
        pltpu.sync_copy(buf0, acc.at[pl.ds(sid * SEG_T + j * CHUNK, CHUNK)])
        return c
    lax.fori_loop(0, SEG_T // CHUNK, _zacc, 0)


def _write_rows(acc, out_hbm, cid, sid):
    def _wout(j, c):
        r = sid * SEG_T + j * CHUNK
        pltpu.sync_copy(acc.at[pl.ds(r, CHUNK)],
                        out_hbm.at[cid, pl.ds(r, CHUNK)])
        return c
    lax.fori_loop(0, SEG_T // CHUNK, _wout, 0)


def _sc_pass1_body(gidx_hbm, sidx_hbm, src_hbm, wvec_hbm,
                   out_hbm, outb_hbm, outd_hbm,
                   gidx_v, sidx_v, buf0, buf1, val0, val1, ones_v, zbuf,
                   acc, saccb, saccd,
                   semr0, semr1, sems0, sems1,
                   semvr0, semvr1, semvs0, semvs1, semb):
    cid = lax.axis_index("c")
    sid = lax.axis_index("s")
    base, nst = _tile_layout(cid, sid)

    _zero_rows(buf0)
    _fill_vec(zbuf, SEG_T, 0.0)
    _fill_vec(ones_v, CHUNK, 1.0)
    _zero_acc(acc, buf0, sid)
    pltpu.sync_copy(zbuf, saccb.at[pl.ds(sid * SEG_T, SEG_T)])
    pltpu.sync_copy(zbuf, saccd.at[pl.ds(sid * SEG_T, SEG_T)])
    plsc.subcore_barrier()

    def _issue_g(j, buf, val, semr, semvr):
        pltpu.async_copy(src_hbm.at[gidx_v.at[j]], buf, semr)
        pltpu.async_copy(wvec_hbm.at[sidx_v.at[j]], val, semvr)

    def _stage(s, c0):
        start = base + s * NHALF
        pltpu.sync_copy(gidx_hbm.at[pl.ds(start, NHALF)], gidx_v)
        pltpu.sync_copy(sidx_hbm.at[pl.ds(start, NHALF)], sidx_v)
        _issue_g(0, buf0, val0, semr0, semvr0)
        _issue_g(1, buf1, val1, semr1, semvr1)

        def _body(i, c):
            j = 2 * i
            pltpu.make_async_copy(src_hbm.at[gidx_v.at[0]], buf0,
                                  semr0).wait()
            pltpu.make_async_copy(wvec_hbm.at[sidx_v.at[0]], val0,
                                  semvr0).wait()
            pltpu.async_copy(buf0, acc.at[sidx_v.at[j]], sems0, add=True)
            pltpu.async_copy(val0, saccd.at[gidx_v.at[j]], semvs0, add=True)
            pltpu.async_copy(ones_v, saccb.at[sidx_v.at[j]], semb, add=True)
            pltpu.make_async_copy(src_hbm.at[gidx_v.at[0]], buf1,
                                  semr1).wait()
            pltpu.make_async_copy(wvec_hbm.at[sidx_v.at[0]], val1,
                                  semvr1).wait()
            pltpu.async_copy(buf1, acc.at[sidx_v.at[j + 1]], sems1, add=True)
            pltpu.async_copy(val1, saccd.at[gidx_v.at[j + 1]], semvs1,
                             add=True)
            pltpu.async_copy(ones_v, saccb.at[sidx_v.at[j + 1]], semb,
                             add=True)
            pltpu.make_async_copy(buf0, acc.at[sidx_v.at[0]], sems0).wait()
            pltpu.make_async_copy(val0, saccd.at[gidx_v.at[0]],
                                  semvs0).wait()
            _issue_g(jnp.minimum(j + 2, LAST), buf0, val0, semr0, semvr0)
            pltpu.make_async_copy(buf1, acc.at[sidx_v.at[0]], sems1).wait()
            pltpu.make_async_copy(val1, saccd.at[gidx_v.at[0]],
                                  semvs1).wait()
            _issue_g(jnp.minimum(j + 3, LAST), buf1, val1, semr1, semvr1)
            pltpu.make_async_copy(ones_v, saccb.at[sidx_v.at[0]],
                                  semb).wait()
            pltpu.make_async_copy(ones_v, saccb.at[sidx_v.at[0]],
                                  semb).wait()
            return c
        lax.fori_loop(0, NHALF // 2, _body, 0)
        pltpu.make_async_copy(src_hbm.at[gidx_v.at[0]], buf0, semr0).wait()
        pltpu.make_async_copy(wvec_hbm.at[sidx_v.at[0]], val0, semvr0).wait()
        pltpu.make_async_copy(src_hbm.at[gidx_v.at[0]], buf1, semr1).wait()
        pltpu.make_async_copy(wvec_hbm.at[sidx_v.at[0]], val1, semvr1).wait()
        return c0
    lax.fori_loop(0, nst, _stage, 0)
    plsc.subcore_barrier()

    _write_rows(acc, out_hbm, cid, sid)
    pltpu.sync_copy(saccb.at[pl.ds(sid * SEG_T, SEG_T)],
                    outb_hbm.at[cid, pl.ds(sid * SEG_T, SEG_T)])
    pltpu.sync_copy(saccd.at[pl.ds(sid * SEG_T, SEG_T)],
                    outd_hbm.at[cid, pl.ds(sid * SEG_T, SEG_T)])


_sc_pass1 = functools.partial(
    pl.kernel,
    out_type=(jax.ShapeDtypeStruct((NCORES, SEG, DF), jnp.float32),
              jax.ShapeDtypeStruct((NCORES, SEG), jnp.float32),
              jax.ShapeDtypeStruct((NCORES, SEG), jnp.float32)),
    mesh=plsc.VectorSubcoreMesh(core_axis_name="c", subcore_axis_name="s"),
    compiler_params=pltpu.CompilerParams(needs_layout_passes=False),
    scratch_types=[
        pltpu.VMEM((NHALF, CHUNK), jnp.int32),
        pltpu.VMEM((NHALF, CHUNK), jnp.int32),
        pltpu.VMEM((CHUNK, DF), jnp.float32),
        pltpu.VMEM((CHUNK, DF), jnp.float32),
        pltpu.VMEM((CHUNK,), jnp.float32),
        pltpu.VMEM((CHUNK,), jnp.float32),
        pltpu.VMEM((CHUNK,), jnp.float32),
        pltpu.VMEM((SEG_T,), jnp.float32),
        pltpu.VMEM_SHARED((SEG, DF), jnp.float32),
        pltpu.VMEM_SHARED((SEG,), jnp.float32),
        pltpu.VMEM_SHARED((SEG,), jnp.float32),
    ] + [pltpu.SemaphoreType.DMA] * 9,
)(_sc_pass1_body)


def _sc_pass2_body(gidx_hbm, sidx_hbm, src_hbm, out_hbm,
                   gidx_v, sidx_v, buf0, buf1,
                   acc, semr0, semr1, sems0, sems1):
    cid = lax.axis_index("c")
    sid = lax.axis_index("s")
    base, nst = _tile_layout(cid, sid)

    _zero_rows(buf0)
    _zero_acc(acc, buf0, sid)
    plsc.subcore_barrier()

    def _stage(s, c0):
        start = base + s * NHALF
        pltpu.sync_copy(gidx_hbm.at[pl.ds(start, NHALF)], gidx_v)
        pltpu.sync_copy(sidx_hbm.at[pl.ds(start, NHALF)], sidx_v)
        pltpu.async_copy(src_hbm.at[gidx_v.at[0]], buf0, semr0)
        pltpu.async_copy(src_hbm.at[gidx_v.at[1]], buf1, semr1)

        def _body(i, c):
            j = 2 * i
            pltpu.make_async_copy(src_hbm.at[gidx_v.at[0]], buf0,
                                  semr0).wait()
            pltpu.async_copy(buf0, acc.at[sidx_v.at[j]], sems0, add=True)
            pltpu.make_async_copy(src_hbm.at[gidx_v.at[0]], buf1,
                                  semr1).wait()
            pltpu.async_copy(buf1, acc.at[sidx_v.at[j + 1]], sems1, add=True)
            pltpu.make_async_copy(buf0, acc.at[sidx_v.at[0]], sems0).wait()
            pltpu.async_copy(src_hbm.at[gidx_v.at[jnp.minimum(j + 2, LAST)]],
                             buf0, semr0)
            pltpu.make_async_copy(buf1, acc.at[sidx_v.at[0]], sems1).wait()
            pltpu.async_copy(src_hbm.at[gidx_v.at[jnp.minimum(j + 3, LAST)]],
                             buf1, semr1)
            return c
        lax.fori_loop(0, NHALF // 2, _body, 0)
        pltpu.make_async_copy(src_hbm.at[gidx_v.at[0]], buf0, semr0).wait()
        pltpu.make_async_copy(src_hbm.at[gidx_v.at[0]], buf1, semr1).wait()
        return c0
    lax.fori_loop(0, nst, _stage, 0)
    plsc.subcore_barrier()

    _write_rows(acc, out_hbm, cid, sid)


_sc_pass2 = functools.partial(
    pl.kernel,
    out_type=jax.ShapeDtypeStruct((NCORES, SEG, DF), jnp.float32),
    mesh=plsc.VectorSubcoreMesh(core_axis_name="c", subcore_axis_name="s"),
    compiler_params=pltpu.CompilerParams(needs_layout_passes=False),
    scratch_types=[
        pltpu.VMEM((NHALF, CHUNK), jnp.int32),
        pltpu.VMEM((NHALF, CHUNK), jnp.int32),
        pltpu.VMEM((CHUNK, DF), jnp.float32),
        pltpu.VMEM((CHUNK, DF), jnp.float32),
        pltpu.VMEM_SHARED((SEG, DF), jnp.float32),
    ] + [pltpu.SemaphoreType.DMA] * 4,
)(_sc_pass2_body)


_CBLK = 2048


def _col_from_tile(s16):
    t = lax.broadcast_in_dim(s16, (16, 128, 128), (0, 2))
    t2 = t.reshape(_CBLK, 128)
    lane = lax.broadcasted_iota(jnp.int32, (_CBLK, 128), 1)
    row = lax.broadcasted_iota(jnp.int32, (_CBLK, 128), 0)
    return jnp.sum(jnp.where(lane == row % 128, t2, 0.0), axis=1,
                   keepdims=True)


def _combine_kernel(p_ref, q_ref, w_ref, w1_ref, b1_ref, o_ref):
    xsum = p_ref[0] + p_ref[1]
    cnt = _col_from_tile(q_ref[0] + q_ref[1])
    binv = jnp.where(cnt > 0, 1.0 / cnt, 0.0)
    mm = jnp.dot(xsum, w1_ref[...], preferred_element_type=jnp.float32)
    o_ref[...] = (mm + cnt * b1_ref[...]) * (binv * w_ref[...])


def _combine(p, q3, w2d, W1, b1):
    return pl.pallas_call(
        _combine_kernel,
        grid=(SEG // _CBLK,),
        in_specs=[pl.BlockSpec((NCORES, _CBLK, DF), lambda i: (0, i, 0)),
                  pl.BlockSpec((NCORES, _CBLK // 128, 128),
                               lambda i: (0, i, 0)),
                  pl.BlockSpec((_CBLK, 1), lambda i: (i, 0)),
                  pl.BlockSpec((DF, DF), lambda i: (0, 0)),
                  pl.BlockSpec((1, DF), lambda i: (0, 0))],
        out_specs=pl.BlockSpec((_CBLK, DF), lambda i: (i, 0)),
        out_shape=jax.ShapeDtypeStruct((E, DF), jnp.float32),
    )(p, q3, w2d, W1, b1)


def _final_kernel(p_ref, q_ref, x_ref, g_ref, b_ref, o_ref):
    osum = p_ref[0] + p_ref[1]
    d = _col_from_tile(q_ref[0] + q_ref[1])
    dinv = jnp.where(d > 0, 1.0 / d, 0.0)
    h = jnp.maximum(x_ref[...] + osum * dinv, 0.0)
    mu = jnp.mean(h, axis=1, keepdims=True)
    var = jnp.mean((h - mu) ** 2, axis=1, keepdims=True)
    o_ref[...] = (h - mu) * lax.rsqrt(var + 1e-5) * g_ref[...] + b_ref[...]


def _final(p, q3, x, gamma, beta):
    return pl.pallas_call(
        _final_kernel,
        grid=(SEG // _CBLK,),
        in_specs=[pl.BlockSpec((NCORES, _CBLK, DF), lambda i: (0, i, 0)),
                  pl.BlockSpec((NCORES, _CBLK // 128, 128),
                               lambda i: (0, i, 0)),
                  pl.BlockSpec((_CBLK, DF), lambda i: (i, 0)),
                  pl.BlockSpec((1, DF), lambda i: (0, 0)),
                  pl.BlockSpec((1, DF), lambda i: (0, 0))],
        out_specs=pl.BlockSpec((_CBLK, DF), lambda i: (i, 0)),
        out_shape=jax.ShapeDtypeStruct((N, DF), jnp.float32),
    )(p, q3, x, gamma, beta)


def kernel(x, hyperedge_index, hyperedge_weight, W1, b1, gamma, beta):
    node = hyperedge_index[0]
    edge = hyperedge_index[1]
    pad = NNZ_PAD - NNZ
    padg = jnp.zeros((pad,), jnp.int32)
    pads = jnp.arange(pad, dtype=jnp.int32) % (SEG - N) + JUNK
    def _lay(a):
        return a.reshape(TOTCH, CHUNK)
    g1 = _lay(jnp.concatenate([node, padg]))
    s1 = _lay(jnp.concatenate([edge, pads]))
    g2 = _lay(jnp.concatenate([edge, padg]))
    s2 = _lay(jnp.concatenate([node, pads]))
    wpad = jnp.concatenate([hyperedge_weight,
                            jnp.zeros((SEG - E,), jnp.float32)])

    p1, qb, qd = _sc_pass1(g1, s1, x, wpad)
    e = _combine(p1, qb.reshape(NCORES, SEG // 128, 128),
                 hyperedge_weight.reshape(E, 1), W1, b1.reshape(1, DF))
    p2 = _sc_pass2(g2, s2, e)
    return _final(p2, qd.reshape(NCORES, SEG // 128, 128),
                  x, gamma.reshape(1, DF), beta.reshape(1, DF))


# speedup vs baseline: 3.8473x; 3.8473x over previous
import functools

import jax
import jax.numpy as jnp
from jax import lax
from jax.experimental import pallas as pl
from jax.experimental.pallas import tpu as pltpu
from jax.experimental.pallas import tpu_sc as plsc

N = 10000
E = 10000
NNZ = 320000
DF = 128

NCORES = 2
NSUB = 16
NW = NCORES * NSUB
CHUNK = 128
NHALF = 40
FAST_CID = 0
CH_F = 80
CH_S = 80
NST_F = CH_F // NHALF
NST_S = CH_S // NHALF
TOTCH = NSUB * (CH_F + CH_S)
NNZ_PAD = TOTCH * CHUNK
SEG = 10240
JUNK = N
SEG_T = SEG // NSUB
LAST = NHALF - 1


def _tile_layout(cid, sid):
    fast = cid == FAST_CID
    base = jnp.where(fast, sid * CH_F, NSUB * CH_F + sid * CH_S)
    nst = jnp.where(fast, NST_F, NST_S)
    return base, nst


def _zero_rows(buf):
    def _zrow(i, c):
        def _zcol(g, c2):
            buf[i, pl.ds(g * 16, 16)] = jnp.zeros((16,), jnp.float32)
            return c2
        return lax.fori_loop(0, DF // 16, _zcol, c)
    lax.fori_loop(0, CHUNK, _zrow, 0)


def _fill_vec(buf, n, value):
    def _zb(g, c):
        buf[pl.ds(g * 16, 16)] = jnp.full((16,), value, jnp.float32)
        return c
    lax.fori_loop(0, n // 16, _zb, 0)


def _zero_acc(acc, buf0, sid):
    def _zacc(j, c):
        pltpu.sync_copy(buf0, acc.at[pl.ds(sid * SEG_T + j * CHUNK, CHUNK)])
        return c
    lax.fori_loop(0, SEG_T // CHUNK, _zacc, 0)


def _write_rows(acc, out_hbm, cid, sid):
    def _wout(j, c):
        r = sid * SEG_T + j * CHUNK
        pltpu.sync_copy(acc.at[pl.ds(r, CHUNK)],
                        out_hbm.at[cid, pl.ds(r, CHUNK)])
        return c
    lax.fori_loop(0, SEG_T // CHUNK, _wout, 0)


def _sc_pass1_body(gidx_hbm, sidx_hbm, src_hbm, wvec_hbm,
                   out_hbm, outb_hbm, outd_hbm,
                   gidx_v, sidx_v, buf0, buf1, val0, val1, ones_v, zbuf,
                   acc, saccb, saccd,
                   semr0, semr1, sems0, sems1,
                   semvr0, semvr1, semvs0, semvs1, semb):
    cid = lax.axis_index("c")
    sid = lax.axis_index("s")
    base, nst = _tile_layout(cid, sid)

    _zero_rows(buf0)
    _fill_vec(zbuf, SEG_T, 0.0)
    _fill_vec(ones_v, CHUNK, 1.0)
    _zero_acc(acc, buf0, sid)
    pltpu.sync_copy(zbuf, saccb.at[pl.ds(sid * SEG_T, SEG_T)])
    pltpu.sync_copy(zbuf, saccd.at[pl.ds(sid * SEG_T, SEG_T)])
    plsc.subcore_barrier()

    def _issue_g(j, buf, val, semr, semvr):
        pltpu.async_copy(src_hbm.at[gidx_v.at[j]], buf, semr)
        pltpu.async_copy(wvec_hbm.at[sidx_v.at[j]], val, semvr)

    def _stage(s, c0):
        start = base + s * NHALF
        pltpu.sync_copy(gidx_hbm.at[pl.ds(start, NHALF)], gidx_v)
        pltpu.sync_copy(sidx_hbm.at[pl.ds(start, NHALF)], sidx_v)
        _issue_g(0, buf0, val0, semr0, semvr0)
        _issue_g(1, buf1, val1, semr1, semvr1)

        def _body(i, c):
            j = 2 * i
            pltpu.make_async_copy(src_hbm.at[gidx_v.at[0]], buf0,
                                  semr0).wait()
            pltpu.make_async_copy(wvec_hbm.at[sidx_v.at[0]], val0,
                                  semvr0).wait()
            pltpu.async_copy(buf0, acc.at[sidx_v.at[j]], sems0, add=True)
            pltpu.async_copy(val0, saccd.at[gidx_v.at[j]], semvs0, add=True)
            pltpu.async_copy(ones_v, saccb.at[sidx_v.at[j]], semb, add=True)
            pltpu.make_async_copy(src_hbm.at[gidx_v.at[0]], buf1,
                                  semr1).wait()
            pltpu.make_async_copy(wvec_hbm.at[sidx_v.at[0]], val1,
                                  semvr1).wait()
            pltpu.async_copy(buf1, acc.at[sidx_v.at[j + 1]], sems1, add=True)
            pltpu.async_copy(val1, saccd.at[gidx_v.at[j + 1]], semvs1,
                             add=True)
            pltpu.async_copy(ones_v, saccb.at[sidx_v.at[j + 1]], semb,
                             add=True)
            pltpu.make_async_copy(buf0, acc.at[sidx_v.at[0]], sems0).wait()
            pltpu.make_async_copy(val0, saccd.at[gidx_v.at[0]],
                                  semvs0).wait()
            _issue_g(jnp.minimum(j + 2, LAST), buf0, val0, semr0, semvr0)
            pltpu.make_async_copy(buf1, acc.at[sidx_v.at[0]], sems1).wait()
            pltpu.make_async_copy(val1, saccd.at[gidx_v.at[0]],
                                  semvs1).wait()
            _issue_g(jnp.minimum(j + 3, LAST), buf1, val1, semr1, semvr1)
            pltpu.make_async_copy(ones_v, saccb.at[sidx_v.at[0]],
                                  semb).wait()
            pltpu.make_async_copy(ones_v, saccb.at[sidx_v.at[0]],
                                  semb).wait()
            return c
        lax.fori_loop(0, NHALF // 2, _body, 0)
        pltpu.make_async_copy(src_hbm.at[gidx_v.at[0]], buf0, semr0).wait()
        pltpu.make_async_copy(wvec_hbm.at[sidx_v.at[0]], val0, semvr0).wait()
        pltpu.make_async_copy(src_hbm.at[gidx_v.at[0]], buf1, semr1).wait()
        pltpu.make_async_copy(wvec_hbm.at[sidx_v.at[0]], val1, semvr1).wait()
        return c0
    lax.fori_loop(0, nst, _stage, 0)
    plsc.subcore_barrier()

    _write_rows(acc, out_hbm, cid, sid)
    pltpu.sync_copy(saccb.at[pl.ds(sid * SEG_T, SEG_T)],
                    outb_hbm.at[cid, pl.ds(sid * SEG_T, SEG_T)])
    pltpu.sync_copy(saccd.at[pl.ds(sid * SEG_T, SEG_T)],
                    outd_hbm.at[cid, pl.ds(sid * SEG_T, SEG_T)])


_sc_pass1 = functools.partial(
    pl.kernel,
    out_type=(jax.ShapeDtypeStruct((NCORES, SEG, DF), jnp.float32),
              jax.ShapeDtypeStruct((NCORES, SEG), jnp.float32),
              jax.ShapeDtypeStruct((NCORES, SEG), jnp.float32)),
    mesh=plsc.VectorSubcoreMesh(core_axis_name="c", subcore_axis_name="s"),
    compiler_params=pltpu.CompilerParams(needs_layout_passes=False),
    scratch_types=[
        pltpu.VMEM((NHALF, CHUNK), jnp.int32),
        pltpu.VMEM((NHALF, CHUNK), jnp.int32),
        pltpu.VMEM((CHUNK, DF), jnp.float32),
        pltpu.VMEM((CHUNK, DF), jnp.float32),
        pltpu.VMEM((CHUNK,), jnp.float32),
        pltpu.VMEM((CHUNK,), jnp.float32),
        pltpu.VMEM((CHUNK,), jnp.float32),
        pltpu.VMEM((SEG_T,), jnp.float32),
        pltpu.VMEM_SHARED((SEG, DF), jnp.float32),
        pltpu.VMEM_SHARED((SEG,), jnp.float32),
        pltpu.VMEM_SHARED((SEG,), jnp.float32),
    ] + [pltpu.SemaphoreType.DMA] * 9,
)(_sc_pass1_body)


def _sc_pass2_body(gidx_hbm, sidx_hbm, src_hbm, out_hbm,
                   gidx_v, sidx_v, buf0, buf1,
                   acc, semr0, semr1, sems0, sems1):
    cid = lax.axis_index("c")
    sid = lax.axis_index("s")
    base, nst = _tile_layout(cid, sid)

    _zero_rows(buf0)
    _zero_acc(acc, buf0, sid)
    plsc.subcore_barrier()

    def _stage(s, c0):
        start = base + s * NHALF
        pltpu.sync_copy(gidx_hbm.at[pl.ds(start, NHALF)], gidx_v)
        pltpu.sync_copy(sidx_hbm.at[pl.ds(start, NHALF)], sidx_v)
        pltpu.async_copy(src_hbm.at[gidx_v.at[0]], buf0, semr0)
        pltpu.async_copy(src_hbm.at[gidx_v.at[1]], buf1, semr1)

        def _body(i, c):
            j = 2 * i
            pltpu.make_async_copy(src_hbm.at[gidx_v.at[0]], buf0,
                                  semr0).wait()
            pltpu.async_copy(buf0, acc.at[sidx_v.at[j]], sems0, add=True)
            pltpu.make_async_copy(src_hbm.at[gidx_v.at[0]], buf1,
                                  semr1).wait()
            pltpu.async_copy(buf1, acc.at[sidx_v.at[j + 1]], sems1, add=True)
            pltpu.make_async_copy(buf0, acc.at[sidx_v.at[0]], sems0).wait()
            pltpu.async_copy(src_hbm.at[gidx_v.at[jnp.minimum(j + 2, LAST)]],
                             buf0, semr0)
            pltpu.make_async_copy(buf1, acc.at[sidx_v.at[0]], sems1).wait()
            pltpu.async_copy(src_hbm.at[gidx_v.at[jnp.minimum(j + 3, LAST)]],
                             buf1, semr1)
            return c
        lax.fori_loop(0, NHALF // 2, _body, 0)
        pltpu.make_async_copy(src_hbm.at[gidx_v.at[0]], buf0, semr0).wait()
        pltpu.make_async_copy(src_hbm.at[gidx_v.at[0]], buf1, semr1).wait()
        return c0
    lax.fori_loop(0, nst, _stage, 0)
    plsc.subcore_barrier()

    _write_rows(acc, out_hbm, cid, sid)


_sc_pass2 = functools.partial(
    pl.kernel,
    out_type=jax.ShapeDtypeStruct((NCORES, SEG, DF), jnp.float32),
    mesh=plsc.VectorSubcoreMesh(core_axis_name="c", subcore_axis_name="s"),
    compiler_params=pltpu.CompilerParams(needs_layout_passes=False),
    scratch_types=[
        pltpu.VMEM((NHALF, CHUNK), jnp.int32),
        pltpu.VMEM((NHALF, CHUNK), jnp.int32),
        pltpu.VMEM((CHUNK, DF), jnp.float32),
        pltpu.VMEM((CHUNK, DF), jnp.float32),
        pltpu.VMEM_SHARED((SEG, DF), jnp.float32),
    ] + [pltpu.SemaphoreType.DMA] * 4,
)(_sc_pass2_body)


_CBLK = 2048


def _col_from_tile(s16):
    t = lax.broadcast_in_dim(s16, (16, 128, 128), (0, 2))
    t2 = t.reshape(_CBLK, 128)
    lane = lax.broadcasted_iota(jnp.int32, (_CBLK, 128), 1)
    row = lax.broadcasted_iota(jnp.int32, (_CBLK, 128), 0)
    return jnp.sum(jnp.where(lane == row % 128, t2, 0.0), axis=1,
                   keepdims=True)


def _combine_kernel(p_ref, q_ref, w_ref, w1_ref, b1_ref, o_ref):
    xsum = p_ref[0] + p_ref[1]
    cnt = _col_from_tile(q_ref[0] + q_ref[1])
    binv = jnp.where(cnt > 0, 1.0 / cnt, 0.0)
    mm = jnp.dot(xsum, w1_ref[...], preferred_element_type=jnp.float32)
    o_ref[...] = (mm + cnt * b1_ref[...]) * (binv * w_ref[...])


def _combine(p, q3, w2d, W1, b1):
    return pl.pallas_call(
        _combine_kernel,
        grid=(SEG // _CBLK,),
        in_specs=[pl.BlockSpec((NCORES, _CBLK, DF), lambda i: (0, i, 0)),
                  pl.BlockSpec((NCORES, _CBLK // 128, 128),
                               lambda i: (0, i, 0)),
                  pl.BlockSpec((_CBLK, 1), lambda i: (i, 0)),
                  pl.BlockSpec((DF, DF), lambda i: (0, 0)),
                  pl.BlockSpec((1, DF), lambda i: (0, 0))],
        out_specs=pl.BlockSpec((_CBLK, DF), lambda i: (i, 0)),
        out_shape=jax.ShapeDtypeStruct((E, DF), jnp.float32),
    )(p, q3, w2d, W1, b1)


def _final_kernel(p_ref, q_ref, x_ref, g_ref, b_ref, o_ref):
    osum = p_ref[0] + p_ref[1]
    d = _col_from_tile(q_ref[0] + q_ref[1])
    dinv = jnp.where(d > 0, 1.0 / d, 0.0)
    h = jnp.maximum(x_ref[...] + osum * dinv, 0.0)
    mu = jnp.mean(h, axis=1, keepdims=True)
    var = jnp.mean((h - mu) ** 2, axis=1, keepdims=True)
    o_ref[...] = (h - mu) * lax.rsqrt(var + 1e-5) * g_ref[...] + b_ref[...]


def _final(p, q3, x, gamma, beta):
    return pl.pallas_call(
        _final_kernel,
        grid=(SEG // _CBLK,),
        in_specs=[pl.BlockSpec((NCORES, _CBLK, DF), lambda i: (0, i, 0)),
                  pl.BlockSpec((NCORES, _CBLK // 128, 128),
                               lambda i: (0, i, 0)),
                  pl.BlockSpec((_CBLK, DF), lambda i: (i, 0)),
                  pl.BlockSpec((1, DF), lambda i: (0, 0)),
                  pl.BlockSpec((1, DF), lambda i: (0, 0))],
        out_specs=pl.BlockSpec((_CBLK, DF), lambda i: (i, 0)),
        out_shape=jax.ShapeDtypeStruct((N, DF), jnp.float32),
    )(p, q3, x, gamma, beta)


def kernel(x, hyperedge_index, hyperedge_weight, W1, b1, gamma, beta):
    node = hyperedge_index[0]
    edge = hyperedge_index[1]
    pad = NNZ_PAD - NNZ
    padg = jnp.arange(pad, dtype=jnp.int32) % N
    pads = jnp.arange(pad, dtype=jnp.int32) % (SEG - N) + JUNK
    def _lay(a):
        return a.reshape(TOTCH, CHUNK)
    g1 = _lay(jnp.concatenate([node, padg]))
    s1 = _lay(jnp.concatenate([edge, pads]))
    g2 = _lay(jnp.concatenate([edge, padg]))
    s2 = _lay(jnp.concatenate([node, pads]))
    wpad = jnp.concatenate([hyperedge_weight,
                            jnp.zeros((SEG - E,), jnp.float32)])

    p1, qb, qd = _sc_pass1(g1, s1, x, wpad)
    e = _combine(p1, qb.reshape(NCORES, SEG // 128, 128),
                 hyperedge_weight.reshape(E, 1), W1, b1.reshape(1, DF))
    p2 = _sc_pass2(g2, s2, e)
    return _final(p2, qd.reshape(NCORES, SEG // 128, 128),
                  x, gamma.reshape(1, DF), beta.reshape(1, DF))


# speedup vs baseline: 3.8526x; 1.0014x over previous
import functools

import jax
import jax.numpy as jnp
from jax import lax
from jax.experimental import pallas as pl
from jax.experimental.pallas import tpu as pltpu
from jax.experimental.pallas import tpu_sc as plsc

N = 10000
E = 10000
NNZ = 320000
DF = 128

NCORES = 2
NSUB = 16
NW = NCORES * NSUB
CHUNK = 128
NHALF = 40
FAST_CID = 0
CH_F = 80
CH_S = 80
NST_F = CH_F // NHALF
NST_S = CH_S // NHALF
TOTCH = NSUB * (CH_F + CH_S)
NNZ_PAD = TOTCH * CHUNK
SEG = 10240
JUNK = N
SEG_T = SEG // NSUB
LAST = NHALF - 1


def _tile_layout(cid, sid):
    fast = cid == FAST_CID
    base = jnp.where(fast, sid * CH_F, NSUB * CH_F + sid * CH_S)
    nst = jnp.where(fast, NST_F, NST_S)
    return base, nst


def _zero_rows(buf):
    def _zrow(i, c):
        def _zcol(g, c2):
            buf[i, pl.ds(g * 16, 16)] = jnp.zeros((16,), jnp.float32)
            return c2
        return lax.fori_loop(0, DF // 16, _zcol, c)
    lax.fori_loop(0, CHUNK, _zrow, 0)


def _fill_vec(buf, n, value):
    def _zb(g, c):
        buf[pl.ds(g * 16, 16)] = jnp.full((16,), value, jnp.float32)
        return c
    lax.fori_loop(0, n // 16, _zb, 0)


def _zero_acc(acc, buf0, sid):
    def _zacc(j, c):
        pltpu.sync_copy(buf0, acc.at[pl.ds(sid * SEG_T + j * CHUNK, CHUNK)])
        return c
    lax.fori_loop(0, SEG_T // CHUNK, _zacc, 0)


def _write_rows(acc, out_hbm, cid, sid):
    def _wout(j, c):
        r = sid * SEG_T + j * CHUNK
        pltpu.sync_copy(acc.at[pl.ds(r, CHUNK)],
                        out_hbm.at[cid, pl.ds(r, CHUNK)])
        return c
    lax.fori_loop(0, SEG_T // CHUNK, _wout, 0)


def _sc_pass1_body(gidx_hbm, sidx_hbm, src_hbm, wvec_hbm,
                   out_hbm, outb_hbm, outd_hbm,
                   gidx_v, sidx_v, buf0, buf1, val0, val1, ones_v, zbuf,
                   acc, saccb, saccd,
                   semr0, semr1, sems0, sems1,
                   semvr0, semvr1, semvs0, semvs1, semb):
    cid = lax.axis_index("c")
    sid = lax.axis_index("s")
    base, nst = _tile_layout(cid, sid)

    _zero_rows(buf0)
    _fill_vec(zbuf, SEG_T, 0.0)
    _fill_vec(ones_v, CHUNK, 1.0)
    _zero_acc(acc, buf0, sid)
    pltpu.sync_copy(zbuf, saccb.at[pl.ds(sid * SEG_T, SEG_T)])
    pltpu.sync_copy(zbuf, saccd.at[pl.ds(sid * SEG_T, SEG_T)])
    plsc.subcore_barrier()

    def _issue_g(j, buf, val, semr, semvr):
        pltpu.async_copy(src_hbm.at[gidx_v.at[j]], buf, semr)
        pltpu.async_copy(wvec_hbm.at[sidx_v.at[j]], val, semvr)

    def _stage(s, c0):
        start = base + s * NHALF
        pltpu.sync_copy(gidx_hbm.at[pl.ds(start, NHALF)], gidx_v)
        pltpu.sync_copy(sidx_hbm.at[pl.ds(start, NHALF)], sidx_v)
        _issue_g(0, buf0, val0, semr0, semvr0)
        _issue_g(1, buf1, val1, semr1, semvr1)

        def _body(i, c):
            j = 2 * i
            pltpu.make_async_copy(src_hbm.at[gidx_v.at[0]], buf0,
                                  semr0).wait()
            pltpu.make_async_copy(wvec_hbm.at[sidx_v.at[0]], val0,
                                  semvr0).wait()
            pltpu.async_copy(buf0, acc.at[sidx_v.at[j]], sems0, add=True)
            pltpu.async_copy(val0, saccd.at[gidx_v.at[j]], semvs0, add=True)
            pltpu.async_copy(ones_v, saccb.at[sidx_v.at[j]], semb, add=True)
            pltpu.make_async_copy(src_hbm.at[gidx_v.at[0]], buf1,
                                  semr1).wait()
            pltpu.make_async_copy(wvec_hbm.at[sidx_v.at[0]], val1,
                                  semvr1).wait()
            pltpu.async_copy(buf1, acc.at[sidx_v.at[j + 1]], sems1, add=True)
            pltpu.async_copy(val1, saccd.at[gidx_v.at[j + 1]], semvs1,
                             add=True)
            pltpu.async_copy(ones_v, saccb.at[sidx_v.at[j + 1]], semb,
                             add=True)
            pltpu.make_async_copy(buf0, acc.at[sidx_v.at[0]], sems0).wait()
            pltpu.make_async_copy(val0, saccd.at[gidx_v.at[0]],
                                  semvs0).wait()
            _issue_g(jnp.minimum(j + 2, LAST), buf0, val0, semr0, semvr0)
            pltpu.make_async_copy(buf1, acc.at[sidx_v.at[0]], sems1).wait()
            pltpu.make_async_copy(val1, saccd.at[gidx_v.at[0]],
                                  semvs1).wait()
            _issue_g(jnp.minimum(j + 3, LAST), buf1, val1, semr1, semvr1)
            pltpu.make_async_copy(ones_v, saccb.at[sidx_v.at[0]],
                                  semb).wait()
            pltpu.make_async_copy(ones_v, saccb.at[sidx_v.at[0]],
                                  semb).wait()
            return c
        lax.fori_loop(0, NHALF // 2, _body, 0)
        pltpu.make_async_copy(src_hbm.at[gidx_v.at[0]], buf0, semr0).wait()
        pltpu.make_async_copy(wvec_hbm.at[sidx_v.at[0]], val0, semvr0).wait()
        pltpu.make_async_copy(src_hbm.at[gidx_v.at[0]], buf1, semr1).wait()
        pltpu.make_async_copy(wvec_hbm.at[sidx_v.at[0]], val1, semvr1).wait()
        return c0
    lax.fori_loop(0, nst, _stage, 0)
    plsc.subcore_barrier()

    _write_rows(acc, out_hbm, cid, sid)
    pltpu.sync_copy(saccb.at[pl.ds(sid * SEG_T, SEG_T)],
                    outb_hbm.at[cid, pl.ds(sid * SEG_T, SEG_T)])
    pltpu.sync_copy(saccd.at[pl.ds(sid * SEG_T, SEG_T)],
                    outd_hbm.at[cid, pl.ds(sid * SEG_T, SEG_T)])


_sc_pass1 = functools.partial(
    pl.kernel,
    out_type=(jax.ShapeDtypeStruct((NCORES, SEG, DF), jnp.float32),
              jax.ShapeDtypeStruct((NCORES, SEG), jnp.float32),
              jax.ShapeDtypeStruct((NCORES, SEG), jnp.float32)),
    mesh=plsc.VectorSubcoreMesh(core_axis_name="c", subcore_axis_name="s"),
    compiler_params=pltpu.CompilerParams(needs_layout_passes=False),
    scratch_types=[
        pltpu.VMEM((NHALF, CHUNK), jnp.int32),
        pltpu.VMEM((NHALF, CHUNK), jnp.int32),
        pltpu.VMEM((CHUNK, DF), jnp.float32),
        pltpu.VMEM((CHUNK, DF), jnp.float32),
        pltpu.VMEM((CHUNK,), jnp.float32),
        pltpu.VMEM((CHUNK,), jnp.float32),
        pltpu.VMEM((CHUNK,), jnp.float32),
        pltpu.VMEM((SEG_T,), jnp.float32),
        pltpu.VMEM_SHARED((SEG, DF), jnp.float32),
        pltpu.VMEM_SHARED((SEG,), jnp.float32),
        pltpu.VMEM_SHARED((SEG,), jnp.float32),
    ] + [pltpu.SemaphoreType.DMA] * 9,
)(_sc_pass1_body)


def _sc_pass2_body(gidx_hbm, sidx_hbm, src_hbm, out_hbm,
                   gidx_v, sidx_v, buf0, buf1,
                   acc, semr0, semr1, sems0, sems1):
    cid = lax.axis_index("c")
    sid = lax.axis_index("s")
    base, nst = _tile_layout(cid, sid)

    _zero_rows(buf0)
    _zero_acc(acc, buf0, sid)
    plsc.subcore_barrier()

    def _stage(s, c0):
        start = base + s * NHALF
        pltpu.sync_copy(gidx_hbm.at[pl.ds(start, NHALF)], gidx_v)
        pltpu.sync_copy(sidx_hbm.at[pl.ds(start, NHALF)], sidx_v)
        pltpu.async_copy(src_hbm.at[gidx_v.at[0]], buf0, semr0)
        pltpu.async_copy(src_hbm.at[gidx_v.at[1]], buf1, semr1)

        def _body(i, c):
            j = 2 * i
            pltpu.make_async_copy(src_hbm.at[gidx_v.at[0]], buf0,
                                  semr0).wait()
            pltpu.async_copy(buf0, acc.at[sidx_v.at[j]], sems0, add=True)
            pltpu.make_async_copy(src_hbm.at[gidx_v.at[0]], buf1,
                                  semr1).wait()
            pltpu.async_copy(buf1, acc.at[sidx_v.at[j + 1]], sems1, add=True)
            pltpu.make_async_copy(buf0, acc.at[sidx_v.at[0]], sems0).wait()
            pltpu.async_copy(src_hbm.at[gidx_v.at[jnp.minimum(j + 2, LAST)]],
                             buf0, semr0)
            pltpu.make_async_copy(buf1, acc.at[sidx_v.at[0]], sems1).wait()
            pltpu.async_copy(src_hbm.at[gidx_v.at[jnp.minimum(j + 3, LAST)]],
                             buf1, semr1)
            return c
        lax.fori_loop(0, NHALF // 2, _body, 0)
        pltpu.make_async_copy(src_hbm.at[gidx_v.at[0]], buf0, semr0).wait()
        pltpu.make_async_copy(src_hbm.at[gidx_v.at[0]], buf1, semr1).wait()
        return c0
    lax.fori_loop(0, nst, _stage, 0)
    plsc.subcore_barrier()

    _write_rows(acc, out_hbm, cid, sid)


_sc_pass2 = functools.partial(
    pl.kernel,
    out_type=jax.ShapeDtypeStruct((NCORES, SEG, DF), jnp.float32),
    mesh=plsc.VectorSubcoreMesh(core_axis_name="c", subcore_axis_name="s"),
    compiler_params=pltpu.CompilerParams(needs_layout_passes=False),
    scratch_types=[
        pltpu.VMEM((NHALF, CHUNK), jnp.int32),
        pltpu.VMEM((NHALF, CHUNK), jnp.int32),
        pltpu.VMEM((CHUNK, DF), jnp.float32),
        pltpu.VMEM((CHUNK, DF), jnp.float32),
        pltpu.VMEM_SHARED((SEG, DF), jnp.float32),
    ] + [pltpu.SemaphoreType.DMA] * 4,
)(_sc_pass2_body)


_CBLK = 2048


def _col_from_tile(s16):
    t = lax.broadcast_in_dim(s16, (16, 128, 128), (0, 2))
    t2 = t.reshape(_CBLK, 128)
    lane = lax.broadcasted_iota(jnp.int32, (_CBLK, 128), 1)
    row = lax.broadcasted_iota(jnp.int32, (_CBLK, 128), 0)
    return jnp.sum(jnp.where(lane == row % 128, t2, 0.0), axis=1,
                   keepdims=True)


def _combine_kernel(p_ref, q_ref, w_ref, w1_ref, b1_ref, o_ref):
    xsum = p_ref[0] + p_ref[1]
    cnt = _col_from_tile(q_ref[0] + q_ref[1])
    binv = jnp.where(cnt > 0, 1.0 / cnt, 0.0)
    mm = jnp.dot(xsum, w1_ref[...], preferred_element_type=jnp.float32)
    o_ref[...] = (mm + cnt * b1_ref[...]) * (binv * w_ref[...])


def _combine(p, q3, w2d, W1, b1):
    return pl.pallas_call(
        _combine_kernel,
        grid=(SEG // _CBLK,),
        in_specs=[pl.BlockSpec((NCORES, _CBLK, DF), lambda i: (0, i, 0)),
                  pl.BlockSpec((NCORES, _CBLK // 128, 128),
                               lambda i: (0, i, 0)),
                  pl.BlockSpec((_CBLK, 1), lambda i: (i, 0)),
                  pl.BlockSpec((DF, DF), lambda i: (0, 0)),
                  pl.BlockSpec((1, DF), lambda i: (0, 0))],
        out_specs=pl.BlockSpec((_CBLK, DF), lambda i: (i, 0)),
        out_shape=jax.ShapeDtypeStruct((SEG, DF), jnp.float32),
    )(p, q3, w2d, W1, b1)


def _final_kernel(p_ref, q_ref, x_ref, g_ref, b_ref, o_ref):
    osum = p_ref[0] + p_ref[1]
    d = _col_from_tile(q_ref[0] + q_ref[1])
    dinv = jnp.where(d > 0, 1.0 / d, 0.0)
    h = jnp.maximum(x_ref[...] + osum * dinv, 0.0)
    mu = jnp.mean(h, axis=1, keepdims=True)
    var = jnp.mean((h - mu) ** 2, axis=1, keepdims=True)
    o_ref[...] = (h - mu) * lax.rsqrt(var + 1e-5) * g_ref[...] + b_ref[...]


def _final(p, q3, x, gamma, beta):
    return pl.pallas_call(
        _final_kernel,
        grid=(SEG // _CBLK,),
        in_specs=[pl.BlockSpec((NCORES, _CBLK, DF), lambda i: (0, i, 0)),
                  pl.BlockSpec((NCORES, _CBLK // 128, 128),
                               lambda i: (0, i, 0)),
                  pl.BlockSpec((_CBLK, DF), lambda i: (i, 0)),
                  pl.BlockSpec((1, DF), lambda i: (0, 0)),
                  pl.BlockSpec((1, DF), lambda i: (0, 0))],
        out_specs=pl.BlockSpec((_CBLK, DF), lambda i: (i, 0)),
        out_shape=jax.ShapeDtypeStruct((N, DF), jnp.float32),
    )(p, q3, x, gamma, beta)


def kernel(x, hyperedge_index, hyperedge_weight, W1, b1, gamma, beta):
    node = hyperedge_index[0]
    edge = hyperedge_index[1]
    pad = NNZ_PAD - NNZ
    padg = jnp.arange(pad, dtype=jnp.int32) % N
    pads = jnp.arange(pad, dtype=jnp.int32) % (SEG - N) + JUNK
    def _lay(a):
        return a.reshape(TOTCH, CHUNK)
    g1 = _lay(jnp.concatenate([node, padg]))
    s1 = _lay(jnp.concatenate([edge, pads]))
    s2 = _lay(jnp.concatenate([node, pads]))
    wpad = jnp.concatenate([hyperedge_weight,
                            jnp.zeros((SEG - E,), jnp.float32)])

    p1, qb, qd = _sc_pass1(g1, s1, x, wpad)
    e = _combine(p1, qb.reshape(NCORES, SEG // 128, 128),
                 wpad.reshape(SEG, 1), W1, b1.reshape(1, DF))
    p2 = _sc_pass2(s1, s2, e)
    return _final(p2, qd.reshape(NCORES, SEG // 128, 128),
                  x, gamma.reshape(1, DF), beta.reshape(1, DF))


# speedup vs baseline: 3.9818x; 1.0335x over previous
import functools

import jax
import jax.numpy as jnp
from jax import lax
from jax.experimental import pallas as pl
from jax.experimental.pallas import tpu as pltpu
from jax.experimental.pallas import tpu_sc as plsc

N = 10000
E = 10000
NNZ = 320000
DF = 128

NCORES = 2
NSUB = 16
NW = NCORES * NSUB
CHUNK = 128
TOTCH = NNZ // CHUNK
CH = 80
NHALF = 40
SEG = 10240
SEG_T = SEG // NSUB


def _tile_chunks(wid):
    base = wid * CH
    nch = jnp.minimum(TOTCH - base, CH)
    nst = jnp.where(nch > NHALF, 2, 1)
    return base, nch, nst


def _stage_window(base, nch, s):
    start = base + s * NHALF
    cnt = jnp.minimum(nch - s * NHALF, NHALF)
    return start, cnt


def _zero_rows(buf):
    def _zrow(i, c):
        def _zcol(g, c2):
            buf[i, pl.ds(g * 16, 16)] = jnp.zeros((16,), jnp.float32)
            return c2
        return lax.fori_loop(0, DF // 16, _zcol, c)
    lax.fori_loop(0, CHUNK, _zrow, 0)


def _fill_vec(buf, n, value):
    def _zb(g, c):
        buf[pl.ds(g * 16, 16)] = jnp.full((16,), value, jnp.float32)
        return c
    lax.fori_loop(0, n // 16, _zb, 0)


def _zero_acc(acc, buf0, sid):
    def _zacc(j, c):
        pltpu.sync_copy(buf0, acc.at[pl.ds(sid * SEG_T + j * CHUNK, CHUNK)])
        return c
    lax.fori_loop(0, SEG_T // CHUNK, _zacc, 0)


def _write_rows(acc, out_hbm, cid, sid):
    def _wout(j, c):
        r = sid * SEG_T + j * CHUNK
        pltpu.sync_copy(acc.at[pl.ds(r, CHUNK)],
                        out_hbm.at[cid, pl.ds(r, CHUNK)])
        return c
    lax.fori_loop(0, SEG_T // CHUNK, _wout, 0)


def _sc_pass1_body(idx_hbm, src_hbm, wvec_hbm,
                   out_hbm, outb_hbm, outd_hbm,
                   gidx_v, sidx_v, buf0, buf1, val0, val1, ones_v, zbuf,
                   acc, saccb, saccd,
                   semr0, semr1, sems0, sems1,
                   semvr0, semvr1, semvs0, semvs1, semb):
    cid = lax.axis_index("c")
    sid = lax.axis_index("s")
    wid = cid * NSUB + sid
    base, nch, nst = _tile_chunks(wid)

    _zero_rows(buf0)
    _fill_vec(zbuf, SEG_T, 0.0)
    _fill_vec(ones_v, CHUNK, 1.0)
    _zero_acc(acc, buf0, sid)
    pltpu.sync_copy(zbuf, saccb.at[pl.ds(sid * SEG_T, SEG_T)])
    pltpu.sync_copy(zbuf, saccd.at[pl.ds(sid * SEG_T, SEG_T)])
    plsc.subcore_barrier()

    def _issue_g(j, buf, val, semr, semvr):
        pltpu.async_copy(src_hbm.at[gidx_v.at[j]], buf, semr)
        pltpu.async_copy(wvec_hbm.at[sidx_v.at[j]], val, semvr)

    def _stage(s, c0):
        start, cnt = _stage_window(base, nch, s)
        pltpu.sync_copy(idx_hbm.at[0, pl.ds(start, NHALF)], gidx_v)
        pltpu.sync_copy(idx_hbm.at[1, pl.ds(start, NHALF)], sidx_v)
        _issue_g(0, buf0, val0, semr0, semvr0)
        _issue_g(1, buf1, val1, semr1, semvr1)

        def _body(i, c):
            j = 2 * i
            pltpu.make_async_copy(src_hbm.at[gidx_v.at[0]], buf0,
                                  semr0).wait()
            pltpu.make_async_copy(wvec_hbm.at[sidx_v.at[0]], val0,
                                  semvr0).wait()
            pltpu.async_copy(buf0, acc.at[sidx_v.at[j]], sems0,
                             add=True)
            pltpu.async_copy(val0, saccd.at[gidx_v.at[j]], semvs0,
                             add=True)
            pltpu.async_copy(ones_v, saccb.at[sidx_v.at[j]], semb,
                             add=True)
            pltpu.make_async_copy(src_hbm.at[gidx_v.at[0]], buf1,
                                  semr1).wait()
            pltpu.make_async_copy(wvec_hbm.at[sidx_v.at[0]], val1,
                                  semvr1).wait()
            pltpu.async_copy(buf1, acc.at[sidx_v.at[j + 1]], sems1,
                             add=True)
            pltpu.async_copy(val1, saccd.at[gidx_v.at[j + 1]], semvs1,
                             add=True)
            pltpu.async_copy(ones_v, saccb.at[sidx_v.at[j + 1]], semb,
                             add=True)
            pltpu.make_async_copy(buf0, acc.at[sidx_v.at[0]], sems0).wait()
            pltpu.make_async_copy(val0, saccd.at[gidx_v.at[0]],
                                  semvs0).wait()
            _issue_g(jnp.minimum(j + 2, cnt - 1), buf0, val0,
                     semr0, semvr0)
            pltpu.make_async_copy(buf1, acc.at[sidx_v.at[0]], sems1).wait()
            pltpu.make_async_copy(val1, saccd.at[gidx_v.at[0]],
                                  semvs1).wait()
            _issue_g(jnp.minimum(j + 3, cnt - 1), buf1, val1,
                     semr1, semvr1)
            pltpu.make_async_copy(ones_v, saccb.at[sidx_v.at[0]],
                                  semb).wait()
            pltpu.make_async_copy(ones_v, saccb.at[sidx_v.at[0]],
                                  semb).wait()
            return c
        lax.fori_loop(0, cnt // 2, _body, 0)
        pltpu.make_async_copy(src_hbm.at[gidx_v.at[0]], buf0, semr0).wait()
        pltpu.make_async_copy(wvec_hbm.at[sidx_v.at[0]], val0, semvr0).wait()
        pltpu.make_async_copy(src_hbm.at[gidx_v.at[0]], buf1, semr1).wait()
        pltpu.make_async_copy(wvec_hbm.at[sidx_v.at[0]], val1, semvr1).wait()
        return c0
    lax.fori_loop(0, nst, _stage, 0)
    plsc.subcore_barrier()

    _write_rows(acc, out_hbm, cid, sid)
    pltpu.sync_copy(saccb.at[pl.ds(sid * SEG_T, SEG_T)],
                    outb_hbm.at[cid, pl.ds(sid * SEG_T, SEG_T)])
    pltpu.sync_copy(saccd.at[pl.ds(sid * SEG_T, SEG_T)],
                    outd_hbm.at[cid, pl.ds(sid * SEG_T, SEG_T)])


_sc_pass1 = functools.partial(
    pl.kernel,
    out_type=(jax.ShapeDtypeStruct((NCORES, SEG, DF), jnp.float32),
              jax.ShapeDtypeStruct((NCORES, SEG), jnp.float32),
              jax.ShapeDtypeStruct((NCORES, SEG), jnp.float32)),
    mesh=plsc.VectorSubcoreMesh(core_axis_name="c", subcore_axis_name="s"),
    compiler_params=pltpu.CompilerParams(needs_layout_passes=False),
    scratch_types=[
        pltpu.VMEM((NHALF, CHUNK), jnp.int32),
        pltpu.VMEM((NHALF, CHUNK), jnp.int32),
        pltpu.VMEM((CHUNK, DF), jnp.float32),
        pltpu.VMEM((CHUNK, DF), jnp.float32),
        pltpu.VMEM((CHUNK,), jnp.float32),
        pltpu.VMEM((CHUNK,), jnp.float32),
        pltpu.VMEM((CHUNK,), jnp.float32),
        pltpu.VMEM((SEG_T,), jnp.float32),
        pltpu.VMEM_SHARED((SEG, DF), jnp.float32),
        pltpu.VMEM_SHARED((SEG,), jnp.float32),
        pltpu.VMEM_SHARED((SEG,), jnp.float32),
    ] + [pltpu.SemaphoreType.DMA] * 9,
)(_sc_pass1_body)


def _sc_pass2_body(idx_hbm, src_hbm, out_hbm,
                   gidx_v, sidx_v, buf0, buf1,
                   acc, semr0, semr1, sems0, sems1):
    cid = lax.axis_index("c")
    sid = lax.axis_index("s")
    wid = cid * NSUB + sid
    base, nch, nst = _tile_chunks(wid)

    _zero_rows(buf0)
    _zero_acc(acc, buf0, sid)
    plsc.subcore_barrier()

    def _stage(s, c0):
        start, cnt = _stage_window(base, nch, s)
        pltpu.sync_copy(idx_hbm.at[1, pl.ds(start, NHALF)], gidx_v)
        pltpu.sync_copy(idx_hbm.at[0, pl.ds(start, NHALF)], sidx_v)
        pltpu.async_copy(src_hbm.at[gidx_v.at[0]], buf0, semr0)
        pltpu.async_copy(src_hbm.at[gidx_v.at[1]], buf1, semr1)

        def _body(i, c):
            j = 2 * i
            pltpu.make_async_copy(src_hbm.at[gidx_v.at[0]], buf0,
                                  semr0).wait()
            pltpu.async_copy(buf0, acc.at[sidx_v.at[j]], sems0,
                             add=True)
            pltpu.make_async_copy(src_hbm.at[gidx_v.at[0]], buf1,
                                  semr1).wait()
            pltpu.async_copy(buf1, acc.at[sidx_v.at[j + 1]], sems1,
                             add=True)
            pltpu.make_async_copy(buf0, acc.at[sidx_v.at[0]], sems0).wait()
            pltpu.async_copy(
                src_hbm.at[gidx_v.at[jnp.minimum(j + 2, cnt - 1)]],
                buf0, semr0)
            pltpu.make_async_copy(buf1, acc.at[sidx_v.at[0]], sems1).wait()
            pltpu.async_copy(
                src_hbm.at[gidx_v.at[jnp.minimum(j + 3, cnt - 1)]],
                buf1, semr1)
            return c
        lax.fori_loop(0, cnt // 2, _body, 0)
        pltpu.make_async_copy(src_hbm.at[gidx_v.at[0]], buf0, semr0).wait()
        pltpu.make_async_copy(src_hbm.at[gidx_v.at[0]], buf1, semr1).wait()
        return c0
    lax.fori_loop(0, nst, _stage, 0)
    plsc.subcore_barrier()

    _write_rows(acc, out_hbm, cid, sid)


_sc_pass2 = functools.partial(
    pl.kernel,
    out_type=jax.ShapeDtypeStruct((NCORES, SEG, DF), jnp.float32),
    mesh=plsc.VectorSubcoreMesh(core_axis_name="c", subcore_axis_name="s"),
    compiler_params=pltpu.CompilerParams(needs_layout_passes=False),
    scratch_types=[
        pltpu.VMEM((NHALF, CHUNK), jnp.int32),
        pltpu.VMEM((NHALF, CHUNK), jnp.int32),
        pltpu.VMEM((CHUNK, DF), jnp.float32),
        pltpu.VMEM((CHUNK, DF), jnp.float32),
        pltpu.VMEM_SHARED((SEG, DF), jnp.float32),
    ] + [pltpu.SemaphoreType.DMA] * 4,
)(_sc_pass2_body)


_CBLK = 2048


def _col_from_tile(s16):
    t = lax.broadcast_in_dim(s16, (16, 128, 128), (0, 2))
    t2 = t.reshape(_CBLK, 128)
    lane = lax.broadcasted_iota(jnp.int32, (_CBLK, 128), 1)
    row = lax.broadcasted_iota(jnp.int32, (_CBLK, 128), 0)
    return jnp.sum(jnp.where(lane == row % 128, t2, 0.0), axis=1,
                   keepdims=True)


def _combine_kernel(p_ref, q_ref, w_ref, w1_ref, b1_ref, o_ref):
    xsum = p_ref[0] + p_ref[1]
    cnt = _col_from_tile(q_ref[0] + q_ref[1])
    binv = jnp.where(cnt > 0, 1.0 / cnt, 0.0)
    mm = jnp.dot(xsum, w1_ref[...], preferred_element_type=jnp.float32)
    o_ref[...] = (mm + cnt * b1_ref[...]) * (binv * w_ref[...])


def _combine(p, q3, w2d, W1, b1):
    return pl.pallas_call(
        _combine_kernel,
        grid=(SEG // _CBLK,),
        in_specs=[pl.BlockSpec((NCORES, _CBLK, DF), lambda i: (0, i, 0)),
                  pl.BlockSpec((NCORES, _CBLK // 128, 128),
                               lambda i: (0, i, 0)),
                  pl.BlockSpec((_CBLK, 1), lambda i: (i, 0)),
                  pl.BlockSpec((DF, DF), lambda i: (0, 0)),
                  pl.BlockSpec((1, DF), lambda i: (0, 0))],
        out_specs=pl.BlockSpec((_CBLK, DF), lambda i: (i, 0)),
        out_shape=jax.ShapeDtypeStruct((SEG, DF), jnp.float32),
    )(p, q3, w2d, W1, b1)


def _final_kernel(p_ref, q_ref, x_ref, g_ref, b_ref, o_ref):
    osum = p_ref[0] + p_ref[1]
    d = _col_from_tile(q_ref[0] + q_ref[1])
    dinv = jnp.where(d > 0, 1.0 / d, 0.0)
    h = jnp.maximum(x_ref[...] + osum * dinv, 0.0)
    mu = jnp.mean(h, axis=1, keepdims=True)
    var = jnp.mean((h - mu) ** 2, axis=1, keepdims=True)
    o_ref[...] = (h - mu) * lax.rsqrt(var + 1e-5) * g_ref[...] + b_ref[...]


def _final(p, q3, x, gamma, beta):
    return pl.pallas_call(
        _final_kernel,
        grid=(SEG // _CBLK,),
        in_specs=[pl.BlockSpec((NCORES, _CBLK, DF), lambda i: (0, i, 0)),
                  pl.BlockSpec((NCORES, _CBLK // 128, 128),
                               lambda i: (0, i, 0)),
                  pl.BlockSpec((_CBLK, DF), lambda i: (i, 0)),
                  pl.BlockSpec((1, DF), lambda i: (0, 0)),
                  pl.BlockSpec((1, DF), lambda i: (0, 0))],
        out_specs=pl.BlockSpec((_CBLK, DF), lambda i: (i, 0)),
        out_shape=jax.ShapeDtypeStruct((N, DF), jnp.float32),
    )(p, q3, x, gamma, beta)


def kernel(x, hyperedge_index, hyperedge_weight, W1, b1, gamma, beta):
    idx3 = jnp.pad(hyperedge_index.reshape(2, TOTCH, CHUNK),
                   ((0, 0), (0, NW * CH - TOTCH), (0, 0)))
    wpad = jnp.concatenate([hyperedge_weight,
                            jnp.zeros((SEG - E,), jnp.float32)])

    p1, qb, qd = _sc_pass1(idx3, x, wpad)
    e = _combine(p1, qb.reshape(NCORES, SEG // 128, 128),
                 wpad.reshape(SEG, 1), W1, b1.reshape(1, DF))
    p2 = _sc_pass2(idx3, e)
    return _final(p2, qd.reshape(NCORES, SEG // 128, 128),
                  x, gamma.reshape(1, DF), beta.reshape(1, DF))
